# baseline (speedup 1.0000x reference)
import functools

import jax
import jax.numpy as jnp
from jax import lax
from jax.experimental import pallas as pl
from jax.experimental.pallas import tpu as pltpu
from jax.experimental.pallas import tpu_sc as plsc

D = 128
L = 16
NS = 16
NC = 2
S = 2048
SU = S // 128
G = 128
R = 13056
ACC_ROWS = 13184
ZROWS = ACC_ROWS // NS
WROWS = R // NS
BN = 1000


@functools.cache
def _make_agg(e_pad: int, n_src: int, n_dst_pad: int):
    assert e_pad % S == 0 and n_dst_pad % (NC * R) == 0
    n_subs = e_pad // S
    n_pass = n_dst_pad // (NC * R)
    mesh = plsc.VectorSubcoreMesh(core_axis_name="c", subcore_axis_name="s")

    @functools.partial(
        pl.kernel,
        out_type=jax.ShapeDtypeStruct((n_dst_pad, D), jnp.float32),
        mesh=mesh,
        scratch_types=(
            pltpu.VMEM((S,), jnp.int32),
            pltpu.VMEM((S,), jnp.int32),
            pltpu.VMEM((G,), jnp.int32),
            pltpu.VMEM((G,), jnp.int32),
            pltpu.VMEM((G, D), jnp.float32),
            pltpu.VMEM_SHARED((ACC_ROWS, D), jnp.float32),
            pltpu.SemaphoreType.DMA,
        ),
    )
    def agg(src_hbm, dst_hbm, x_hbm, z128_hbm,
            sum_out,
            sv, dv, dbuf, sbuf, rows, acc, sem):
        cid = lax.axis_index("c")
        sid = lax.axis_index("s")
        n_mine = (n_subs - sid + NS - 1) // NS

        def pass_body(p, carry):
            lo = (p * NC + cid) * R
            zbase = sid * ZROWS
            for t in range(ZROWS // G):
                pltpu.sync_copy(z128_hbm, acc.at[pl.ds(zbase + t * G, G)])
            rem_z = ZROWS % G
            if rem_z:
                zb2 = zbase + (ZROWS // G) * G
                pltpu.sync_copy(z128_hbm.at[pl.ds(0, rem_z)],
                                acc.at[pl.ds(zb2, rem_z)])
            plsc.subcore_barrier()

            lo_v = jnp.full((L,), lo, jnp.int32)
            hi_v = lo_v + R
            dump_v = jnp.full((L,), R, jnp.int32)

            def sub_body(j, c2):
                ebase = (j * NS + sid) * S
                pltpu.sync_copy(src_hbm.at[pl.ds(ebase, S)], sv)
                pltpu.sync_copy(dst_hbm.at[pl.ds(ebase, S)], dv)

                def unit_body(u, c3):
                    ub = u * G
                    for jj in range(G // L):
                        d = dv[pl.ds(ub + jj * L, L)]
                        m = (d >= lo_v) & (d < hi_v)
                        dbuf[pl.ds(jj * L, L)] = jnp.where(m, d - lo_v,
                                                           dump_v)
                        sbuf[pl.ds(jj * L, L)] = sv[pl.ds(ub + jj * L, L)]
                    pltpu.async_copy(x_hbm.at[sbuf], rows, sem).wait()
                    pltpu.sync_copy(rows, acc.at[dbuf], add=True)
                    return c3

                lax.fori_loop(0, SU, unit_body, 0)
                return c2

            lax.fori_loop(0, n_mine, sub_body, 0)
            plsc.subcore_barrier()

            wb = sid * WROWS
            ob = (p * NC + cid) * R + wb
            pltpu.sync_copy(acc.at[pl.ds(wb, WROWS)],
                            sum_out.at[pl.ds(ob, WROWS)])
            plsc.subcore_barrier()
            return carry

        lax.fori_loop(0, n_pass, pass_body, 0)

    return agg


@functools.cache
def _make_cnt(e_pad: int, n_flat: int):
    nw = NC * NS
    assert e_pad % (nw * S) == 0 and n_flat % G == 0
    n_subs = e_pad // S
    nm = n_subs // nw
    mesh = plsc.VectorSubcoreMesh(core_axis_name="c", subcore_axis_name="s")

    @functools.partial(
        pl.kernel,
        out_type=jax.ShapeDtypeStruct((nw, n_flat), jnp.float32),
        mesh=mesh,
        scratch_types=(
            pltpu.VMEM((nm * S,), jnp.int32),
            pltpu.VMEM((n_flat + 2 * L,), jnp.float32),
        ),
    )
    def cnt(dst_hbm, cnt_out, dv, cl):
        cid = lax.axis_index("c")
        sid = lax.axis_index("s")
        w = cid * NS + sid
        zf = jnp.zeros((L,), jnp.float32)

        def zero_body(i, c0):
            cl[pl.ds(i * L, L)] = zf
            return c0

        lax.fori_loop(0, n_flat // L + 2, zero_body, 0)
        for j in range(nm):
            pltpu.sync_copy(dst_hbm.at[pl.ds((j * nw + w) * S, S)],
                            dv.at[pl.ds(j * S, S)])
        hot = jnp.where(lax.iota(jnp.int32, L) == 0, 1.0, 0.0)
        sink = jnp.full((L,), n_flat, jnp.int32)

        def vec_body(i, c3):
            d = dv[pl.ds(i * L, L)]
            ds_ = jnp.where(d >= 0, d, sink)
            for j in range(L):
                dj = ds_[j]
                cl[pl.ds(dj, L)] = cl[pl.ds(dj, L)] + hot
            return c3

        lax.fori_loop(0, nm * (S // L), vec_body, 0)
        pltpu.sync_copy(cl.at[pl.ds(0, n_flat)], cnt_out.at[w])

    return cnt


def _cntred_body(p_ref, o_ref):
    s = jnp.sum(p_ref[...], axis=0)
    o_ref[...] = 1.0 / jnp.maximum(s, 1.0)


@functools.cache
def _make_cntred(n_flat: int):
    cpt = n_flat // G
    cb = max(d for d in range(8, 129, 8) if cpt % d == 0)
    nw = NC * NS
    return pl.pallas_call(
        _cntred_body,
        grid=(cpt // cb,),
        in_specs=[pl.BlockSpec((nw, cb, G), lambda i: (0, i, 0))],
        out_specs=pl.BlockSpec((cb, G), lambda i: (i, 0)),
        out_shape=jax.ShapeDtypeStruct((cpt, G), jnp.float32),
    )


def _sage_block(sum_blk, inv_blk, x_blk, wl, wr, b):
    mean = sum_blk * inv_blk
    t = (lax.dot_general(mean, wl, (((1,), (1,)), ((), ())),
                         preferred_element_type=jnp.float32)
         + b
         + lax.dot_general(x_blk, wr, (((1,), (1,)), ((), ())),
                           preferred_element_type=jnp.float32))
    n2 = jnp.sum(t * t, axis=1, keepdims=True)
    return t * lax.rsqrt(jnp.maximum(n2, 1e-24))


def _dense1_body(sum_ref, inv_ref, x_ref, wl_ref, wr_ref, b_ref, o_ref):
    t = _sage_block(sum_ref[...], inv_ref[...], x_ref[...],
                    wl_ref[...], wr_ref[...], b_ref[...])
    o_ref[...] = jnp.where(t > 0, t, 0.01 * t)


def _dense2_body(s1_ref, i1_ref, s2_ref, i2_ref, x_ref,
                 wl1_ref, wr1_ref, b1_ref, wl2_ref, wr2_ref, b2_ref, o_ref):
    x_blk = x_ref[...]
    t = (_sage_block(s1_ref[...], i1_ref[...], x_blk,
                     wl1_ref[...], wr1_ref[...], b1_ref[...])
         + _sage_block(s2_ref[...], i2_ref[...], x_blk,
                       wl2_ref[...], wr2_ref[...], b2_ref[...]))
    o_ref[...] = jnp.where(t > 0, t, 0.01 * t)


@functools.cache
def _make_dense(n: int, two: bool):
    assert n % BN == 0
    row = pl.BlockSpec((BN, D), lambda i: (i, 0))
    cnt = pl.BlockSpec((BN, 1), lambda i: (i, 0))
    w = pl.BlockSpec((D, D), lambda i: (0, 0))
    b = pl.BlockSpec((1, D), lambda i: (0, 0))
    if two:
        in_specs = [row, cnt, row, cnt, row, w, w, b, w, w, b]
        body = _dense2_body
    else:
        in_specs = [row, cnt, row, w, w, b]
        body = _dense1_body
    return pl.pallas_call(
        body,
        grid=(n // BN,),
        in_specs=in_specs,
        out_specs=row,
        out_shape=jax.ShapeDtypeStruct((n, D), jnp.float32),
    )


def _pad_edges(ei):
    e = ei.shape[1]
    ep = -(-e // (NC * NS * S)) * (NC * NS * S)
    src, dst = ei[0], ei[1]
    if ep != e:
        src = jnp.concatenate([src, jnp.zeros((ep - e,), jnp.int32)])
        dst = jnp.concatenate([dst, jnp.full((ep - e,), -1, jnp.int32)])
    return src, dst


def kernel(x_ip, x_con, ei_ip_ip, ei_con_src, ei_con_dst, ei_ip_con,
           ei_con_ip, Wl, Wr, bl):
    z128 = jnp.zeros((G, D), jnp.float32)

    eII = _pad_edges(ei_ip_ip)
    eCS = _pad_edges(ei_con_src)
    eCD = _pad_edges(ei_con_dst)
    eIC = _pad_edges(ei_ip_con)
    eCI = _pad_edges(ei_con_ip)

    def agg(e, x, n_dst):
        n_dst_pad = -(-n_dst // (NC * R)) * (NC * R)
        f = _make_agg(e[0].shape[0], x.shape[0], n_dst_pad)
        summ = f(e[0], e[1], x, z128)
        n_flat = -(-n_dst // (NS * G)) * (NS * G)
        fc = _make_cnt(e[0].shape[0], n_flat)
        parts = fc(e[1])
        inv2d = _make_cntred(n_flat)(parts.reshape(NC * NS, n_flat // G, G))
        inv = inv2d.reshape(n_flat, 1)
        return summ, inv

    def dense1(a, x, i):
        return _make_dense(x.shape[0], False)(
            a[0], a[1], x, Wl[i], Wr[i], bl[i][None, :])

    def dense2(a1, a2, x, i1, i2):
        return _make_dense(x.shape[0], True)(
            a1[0], a1[1], a2[0], a2[1], x,
            Wl[i1], Wr[i1], bl[i1][None, :], Wl[i2], Wr[i2], bl[i2][None, :])

    xi, xc = x_ip, x_con
    n_ip, n_con = x_ip.shape[0], x_con.shape[0]
    idx = 0
    for _ in range(2):
        aII = agg(eII, xi, n_ip)
        aCS = agg(eCS, xc, n_con)
        aCD = agg(eCD, xc, n_con)
        xi2 = dense1(aII, xi, idx)
        xc2 = dense2(aCS, aCD, xc, idx + 1, idx + 2)
        idx += 3
        aIC = agg(eIC, xi2, n_con)
        aCI = agg(eCI, xc2, n_ip)
        xc = dense1(aIC, xc2, idx)
        xi = dense1(aCI, xi2, idx + 1)
        idx += 2
    return (xi, xc)
